# baseline (speedup 1.0000x reference)
import functools

import jax
import jax.numpy as jnp
from jax import lax
from jax.experimental import pallas as pl
from jax.experimental.pallas import tpu as pltpu
from jax.experimental.pallas import tpu_sc as plsc

_B, _S, _E = 4, 8192, 768
_NW = 32
_RPW = _S // _NW
_SUB = 32
_NCH = _RPW // _SUB
_NST = _NCH * _B
_CHW = _SUB * _E
_NTB = 3

_mesh = plsc.VectorSubcoreMesh(core_axis_name="c", subcore_axis_name="s")


@functools.partial(
    pl.kernel,
    mesh=_mesh,
    out_type=jax.ShapeDtypeStruct((_B * _S * _E,), jnp.float32),
    scratch_types=(
        [pltpu.VMEM((_CHW,), jnp.float32) for _ in range(2)]
        + [pltpu.VMEM((_CHW,), jnp.float32) for _ in range(_NTB)]
        + [
            pltpu.SemaphoreType.DMA,
            pltpu.SemaphoreType.DMA,
            pltpu.SemaphoreType.DMA,
        ]
    ),
)
def _sc_add(tok_hbm, pos_hbm, out_hbm, pbuf0, pbuf1, tbuf0, tbuf1, tbuf2,
            psem, tsem, osem):
    pbuf = [pbuf0, pbuf1]
    tbuf = [tbuf0, tbuf1, tbuf2]
    wid = lax.axis_index("s") * 2 + lax.axis_index("c")
    s0 = wid * _RPW

    def tok_off(t):
        c, b = divmod(t, _B)
        return (b * _S + s0 + c * _SUB) * _E

    def load_tok(t):
        return pltpu.async_copy(
            tok_hbm.at[pl.ds(tok_off(t), _CHW)], tbuf[t % _NTB], tsem)

    def load_pos(c):
        return pltpu.async_copy(
            pos_hbm.at[pl.ds((s0 + c * _SUB) * _E, _CHW)], pbuf[c % 2], psem)

    pos_d = [load_pos(0)]
    tok_d = [load_tok(t) for t in range(_NTB - 1)]
    store_d = []

    for t in range(_NST):
        c, b = divmod(t, _B)
        if b == 0:
            pos_d.pop(0).wait()
            if c + 1 < _NCH:
                pos_d.append(load_pos(c + 1))
        tok_d.pop(0).wait()
        tb = tbuf[t % _NTB]
        pb = pbuf[c % 2]

        @plsc.parallel_loop(0, _CHW, 16, unroll=8)
        def _(i):
            sl = pl.ds(i, 16)
            plsc.addupdate(tb.at[sl], pb[sl])

        store_d.append(
            pltpu.async_copy(tb, out_hbm.at[pl.ds(tok_off(t), _CHW)], osem))
        if t + _NTB - 1 < _NST:
            if len(store_d) > 1:
                store_d.pop(0).wait()
            tok_d.append(load_tok(t + _NTB - 1))

    for d in store_d:
        d.wait()


def kernel(token_embedding, src_mask, pos_embedding):
    B, S, E = token_embedding.shape
    out = _sc_add(token_embedding.reshape(-1), pos_embedding[:S].reshape(-1))
    return out.reshape(B, S, E)


# speedup vs baseline: 3.0326x; 3.0326x over previous
import functools

import jax
import jax.numpy as jnp
from jax import lax
from jax.experimental import pallas as pl
from jax.experimental.pallas import tpu as pltpu
from jax.experimental.pallas import tpu_sc as plsc

_B, _S, _E = 4, 8192, 768
_NW = 32
_RPW = _S // _NW
_SUB = 32
_NCH = _RPW // _SUB
_NST = _NCH * _B
_NTB = 3

_mesh = plsc.VectorSubcoreMesh(core_axis_name="c", subcore_axis_name="s")


@functools.partial(
    pl.kernel,
    mesh=_mesh,
    out_type=jax.ShapeDtypeStruct((_B * _S, _E), jnp.float32),
    scratch_types=(
        [pltpu.VMEM((_SUB, _E), jnp.float32) for _ in range(2)]
        + [pltpu.VMEM((_SUB, _E), jnp.float32) for _ in range(_NTB)]
        + [
            pltpu.SemaphoreType.DMA,
            pltpu.SemaphoreType.DMA,
            pltpu.SemaphoreType.DMA,
        ]
    ),
)
def _sc_add(tok_hbm, pos_hbm, out_hbm, pbuf0, pbuf1, tbuf0, tbuf1, tbuf2,
            psem, tsem, osem):
    pbuf = [pbuf0, pbuf1]
    tbuf = [tbuf0, tbuf1, tbuf2]
    wid = lax.axis_index("s") * 2 + lax.axis_index("c")
    s0 = wid * _RPW

    def tok_row(t):
        c, b = divmod(t, _B)
        return b * _S + s0 + c * _SUB

    def load_tok(t):
        return pltpu.async_copy(
            tok_hbm.at[pl.ds(tok_row(t), _SUB)], tbuf[t % _NTB], tsem)

    def load_pos(c):
        return pltpu.async_copy(
            pos_hbm.at[pl.ds(s0 + c * _SUB, _SUB)], pbuf[c % 2], psem)

    pos_d = [load_pos(0)]
    tok_d = [load_tok(t) for t in range(_NTB - 1)]
    store_d = []

    for t in range(_NST):
        c, b = divmod(t, _B)
        if b == 0:
            pos_d.pop(0).wait()
            if c + 1 < _NCH:
                pos_d.append(load_pos(c + 1))
        tok_d.pop(0).wait()
        tb = tbuf[t % _NTB]
        pb = pbuf[c % 2]

        def row_body(r, carry):
            @plsc.parallel_loop(0, _E, 16, unroll=8)
            def _row_add(i):
                sl = pl.ds(i, 16)
                plsc.addupdate(tb.at[r, sl], pb[r, sl])
            return carry

        lax.fori_loop(0, _SUB, row_body, 0)

        store_d.append(
            pltpu.async_copy(tb, out_hbm.at[pl.ds(tok_row(t), _SUB)], osem))
        if t + _NTB - 1 < _NST:
            if len(store_d) > 1:
                store_d.pop(0).wait()
            tok_d.append(load_tok(t + _NTB - 1))

    for d in store_d:
        d.wait()


def kernel(token_embedding, src_mask, pos_embedding):
    B, S, E = token_embedding.shape
    out = _sc_add(token_embedding.reshape(B * S, E), pos_embedding[:S])
    return out.reshape(B, S, E)
